# baseline (speedup 1.0000x reference)
import functools

import jax
import jax.numpy as jnp
from jax import lax
from jax.experimental import pallas as pl
from jax.experimental.pallas import tpu as pltpu
from jax.experimental.pallas import tpu_sc as plsc

_B, _S, _D, _V = 4, 4096, 2048, 4096

_RB = 256
_NB = _V // _RB


def _table_body(dir_ref, scale_ref, out_ref):
    i = pl.program_id(0)

    @pl.when(i < _NB)
    def _compute():
        d = dir_ref[...]
        s = scale_ref[...]
        norm = jnp.sqrt(jnp.sum(d * d, axis=1, keepdims=True))
        out_ref[...] = d * (s[:, None] / jnp.maximum(norm, 1e-8))

    @pl.when(i == _NB)
    def _zero():
        out_ref[...] = jnp.zeros_like(out_ref)


def _build_table(direction_VD, scale_V):
    return pl.pallas_call(
        _table_body,
        grid=(_NB + 1,),
        in_specs=[
            pl.BlockSpec((_RB, _D), lambda i: (jnp.minimum(i, _NB - 1), 0)),
            pl.BlockSpec((_RB,), lambda i: (jnp.minimum(i, _NB - 1),)),
        ],
        out_specs=pl.BlockSpec((_RB, _D), lambda i: (i, 0)),
        out_shape=jax.ShapeDtypeStruct(((_NB + 1) * _RB, _D), jnp.float32),
    )(direction_VD, scale_V)


_NC, _NS, _L = 2, 16, 16
_NW = _NC * _NS
_TOK = _B * _S
_TPW = _TOK // _NW
_T = 16
_NCH = _TPW // _T


def _sc_body(hid_hbm, steer_hbm, ptr_hbm, out_hbm, idx_v, hid_v, rows_v,
             gsem, hsem):
    wid = lax.axis_index("s") * _NC + lax.axis_index("c")
    base = wid * _TPW
    pltpu.sync_copy(ptr_hbm.at[pl.ds(wid * _NCH, _NCH), :], idx_v)

    def chunk(c, carry):
        tok0 = base + c * _T
        hcp = pltpu.async_copy(hid_hbm.at[pl.ds(tok0, _T), :], hid_v, hsem)
        gcp = pltpu.async_copy(steer_hbm.at[idx_v.at[c]], rows_v, gsem)
        hcp.wait()
        gcp.wait()

        def tokadd(i, c2):
            def vecadd(j, c3):
                plsc.addupdate(hid_v.at[i, pl.ds(j * _L, _L)],
                               rows_v[i, pl.ds(j * _L, _L)])
                return c3
            return lax.fori_loop(0, _D // _L, vecadd, c2)

        lax.fori_loop(0, _T, tokadd, 0)
        pltpu.sync_copy(hid_v, out_hbm.at[pl.ds(tok0, _T), :])
        return carry

    lax.fori_loop(0, _NCH, chunk, 0)


_sc_apply = functools.partial(
    pl.kernel,
    out_type=jax.ShapeDtypeStruct((_TOK, _D), jnp.float32),
    mesh=plsc.VectorSubcoreMesh(core_axis_name="c", subcore_axis_name="s"),
    scratch_types=[
        pltpu.VMEM((_NCH, _L), jnp.int32),
        pltpu.VMEM((_T, _D), jnp.float32),
        pltpu.VMEM((_T, _D), jnp.float32),
        pltpu.SemaphoreType.DMA,
        pltpu.SemaphoreType.DMA,
    ],
)(_sc_body)


@jax.jit
def kernel(hidden_BSD, direction_VD, scale_V, zero_vec_D, vec_ptrs_BS):
    del zero_vec_D
    steer = _build_table(direction_VD, scale_V)
    hid2d = hidden_BSD.reshape(_TOK, _D)
    ptr2d = vec_ptrs_BS.reshape(_TOK // _L, _L)
    out = _sc_apply(hid2d, steer, ptr2d)
    return out.reshape(_B, _S, _D)


# speedup vs baseline: 2.6544x; 2.6544x over previous
import functools

import jax
import jax.numpy as jnp
from jax import lax
from jax.experimental import pallas as pl
from jax.experimental.pallas import tpu as pltpu
from jax.experimental.pallas import tpu_sc as plsc

_B, _S, _D, _V = 4, 4096, 2048, 4096

_RB = 256
_NB = _V // _RB


def _table_body(dir_ref, scale_ref, out_ref):
    i = pl.program_id(0)

    @pl.when(i < _NB)
    def _compute():
        d = dir_ref[...]
        s = scale_ref[...]
        norm = jnp.sqrt(jnp.sum(d * d, axis=1, keepdims=True))
        out_ref[...] = d * (s[:, None] / jnp.maximum(norm, 1e-8))

    @pl.when(i == _NB)
    def _zero():
        out_ref[...] = jnp.zeros_like(out_ref)


def _build_table(direction_VD, scale_V):
    return pl.pallas_call(
        _table_body,
        grid=(_NB + 1,),
        in_specs=[
            pl.BlockSpec((_RB, _D), lambda i: (jnp.minimum(i, _NB - 1), 0)),
            pl.BlockSpec((_RB,), lambda i: (jnp.minimum(i, _NB - 1),)),
        ],
        out_specs=pl.BlockSpec((_RB, _D), lambda i: (i, 0)),
        out_shape=jax.ShapeDtypeStruct(((_NB + 1) * _RB, _D), jnp.float32),
    )(direction_VD, scale_V)


_NC, _NS, _L = 2, 16, 16
_NW = _NC * _NS
_TOK = _B * _S
_TPW = _TOK // _NW
_T = 8
_NCH = _TPW // _T
_NBUF = 2


def _sc_body(hid_hbm, steer_hbm, ptr_hbm, out_hbm, idx_v, hid_v, rows_v,
             out_v, hsem, gsem, osem):
    wid = lax.axis_index("s") * _NC + lax.axis_index("c")
    base = wid * _TPW
    pltpu.sync_copy(ptr_hbm.at[pl.ds(wid * _NCH, _NCH), :], idx_v)

    def issue_in(c, b):
        tok0 = base + c * _T
        pltpu.async_copy(hid_hbm.at[pl.ds(tok0, _T), :], hid_v[b],
                         hsem.at[b])
        pltpu.async_copy(steer_hbm.at[idx_v.at[c]], rows_v[b], gsem.at[b])

    def wait_in(c, b):
        tok0 = base + c * _T
        pltpu.make_async_copy(hid_hbm.at[pl.ds(tok0, _T), :], hid_v[b],
                              hsem.at[b]).wait()
        pltpu.make_async_copy(steer_hbm.at[idx_v.at[c]], rows_v[b],
                              gsem.at[b]).wait()

    def wait_out(b):
        pltpu.make_async_copy(out_v[b], out_hbm.at[pl.ds(base, _T), :],
                              osem.at[b]).wait()

    issue_in(0, 0)
    issue_in(1, 1)

    def group(c2, carry):
        for b in range(_NBUF):
            c = c2 * _NBUF + b
            wait_in(c, b)
            @pl.when(c >= _NBUF)
            def _():
                wait_out(b)
            for t in range(_T):
                @plsc.parallel_loop(0, _D // _L, unroll=8)
                def _add(j):
                    sl = pl.ds(j * _L, _L)
                    out_v[b][t, sl] = hid_v[b][t, sl] + rows_v[b][t, sl]
            tok0 = base + c * _T
            pltpu.async_copy(out_v[b], out_hbm.at[pl.ds(tok0, _T), :],
                             osem.at[b])
            @pl.when(c + _NBUF < _NCH)
            def _():
                issue_in(c + _NBUF, b)
        return carry

    lax.fori_loop(0, _NCH // _NBUF, group, 0)
    for b in range(_NBUF):
        wait_out(b)


_sc_apply = functools.partial(
    pl.kernel,
    out_type=jax.ShapeDtypeStruct((_TOK, _D), jnp.float32),
    mesh=plsc.VectorSubcoreMesh(core_axis_name="c", subcore_axis_name="s"),
    scratch_types=[
        pltpu.VMEM((_NCH, _T), jnp.int32),
        [pltpu.VMEM((_T, _D), jnp.float32) for _ in range(_NBUF)],
        [pltpu.VMEM((_T, _D), jnp.float32) for _ in range(_NBUF)],
        [pltpu.VMEM((_T, _D), jnp.float32) for _ in range(_NBUF)],
        pltpu.SemaphoreType.DMA((_NBUF,)),
        pltpu.SemaphoreType.DMA((_NBUF,)),
        pltpu.SemaphoreType.DMA((_NBUF,)),
    ],
)(_sc_body)


@jax.jit
def kernel(hidden_BSD, direction_VD, scale_V, zero_vec_D, vec_ptrs_BS):
    del zero_vec_D
    steer = _build_table(direction_VD, scale_V)
    hid2d = hidden_BSD.reshape(_TOK, _D)
    ptr2d = vec_ptrs_BS.reshape(_TOK // _T, _T)
    out = _sc_apply(hid2d, steer, ptr2d)
    return out.reshape(_B, _S, _D)
